# baseline (speedup 1.0000x reference)
import functools

import jax
import jax.numpy as jnp
from jax import lax
from jax.experimental import pallas as pl
from jax.experimental.pallas import tpu as pltpu
from jax.experimental.pallas import tpu_sc as plsc

EMB = 64
SCALE = 8.0

NC = 2
NS = 16
NW = NC * NS

SUB = 128
CHUNK = 512
NSUB = CHUNK // SUB


@functools.partial(jax.jit, static_argnums=(2,))
def _lookup(tok2d, table, n):
    per_w = n // NW
    n_chunks = per_w // CHUNK
    tok_rows_per_w = per_w // SUB

    mesh = plsc.VectorSubcoreMesh(core_axis_name="c", subcore_axis_name="s")

    @functools.partial(
        pl.kernel,
        mesh=mesh,
        out_type=jax.ShapeDtypeStruct((n, EMB), jnp.float32),
        compiler_params=pltpu.CompilerParams(use_tc_tiling_on_sc=False),
        scratch_types=[
            pltpu.VMEM((NSUB, SUB), jnp.int32),
            pltpu.VMEM((CHUNK, EMB), jnp.float32),
            pltpu.SemaphoreType.DMA,
        ],
    )
    def body(tok_hbm, table_hbm, out_hbm, idx_v, rows_v, sem):
        c = lax.axis_index("c")
        s = lax.axis_index("s")
        wid = s * NC + c
        row_base = wid * per_w
        tok_base = wid * tok_rows_per_w

        def chunk_body(i, carry):
            pltpu.sync_copy(tok_hbm.at[pl.ds(tok_base + i * NSUB, NSUB)], idx_v)
            copies = [
                pltpu.async_copy(
                    table_hbm.at[idx_v.at[j]],
                    rows_v.at[pl.ds(j * SUB, SUB)],
                    sem,
                )
                for j in range(NSUB)
            ]
            for cp in copies:
                cp.wait()

            def scale_row(r, carry2):
                for col in range(0, EMB, 16):
                    rows_v[r, pl.ds(col, 16)] = rows_v[r, pl.ds(col, 16)] * SCALE
                return carry2

            lax.fori_loop(0, CHUNK, scale_row, 0, unroll=2)

            pltpu.sync_copy(
                rows_v, out_hbm.at[pl.ds(row_base + i * CHUNK, CHUNK)]
            )
            return carry

        lax.fori_loop(0, n_chunks, chunk_body, 0)

    return body(tok2d, table)


def kernel(tokens, table):
    b, l = tokens.shape
    n = b * l
    tok2d = tokens.reshape(n // SUB, SUB).astype(jnp.int32)
    out = _lookup(tok2d, table, n)
    return out.reshape(b, l, EMB)


# speedup vs baseline: 1.0883x; 1.0883x over previous
import functools

import jax
import jax.numpy as jnp
from jax import lax
from jax.experimental import pallas as pl
from jax.experimental.pallas import tpu as pltpu
from jax.experimental.pallas import tpu_sc as plsc

EMB = 64
SCALE = 8.0

NC = 2
NS = 16
NW = NC * NS

SUB = 128
CHUNK = 512
NSUB = CHUNK // SUB


@functools.partial(jax.jit, static_argnums=(2,))
def _lookup(tok2d, table, n):
    per_w = n // NW
    n_chunks = per_w // CHUNK
    n_pairs = n_chunks // 2
    tok_rows_per_w = per_w // SUB

    mesh = plsc.VectorSubcoreMesh(core_axis_name="c", subcore_axis_name="s")

    @functools.partial(
        pl.kernel,
        mesh=mesh,
        out_type=jax.ShapeDtypeStruct((n, EMB), jnp.float32),
        compiler_params=pltpu.CompilerParams(use_tc_tiling_on_sc=False),
        scratch_types=[
            pltpu.VMEM((tok_rows_per_w, SUB), jnp.int32),
            pltpu.VMEM((CHUNK, EMB), jnp.float32),
            pltpu.VMEM((CHUNK, EMB), jnp.float32),
            pltpu.SemaphoreType.DMA,
            pltpu.SemaphoreType.DMA,
            pltpu.SemaphoreType.DMA,
            pltpu.SemaphoreType.DMA,
        ],
    )
    def body(tok_hbm, table_hbm, out_hbm, idx_all, rows0, rows1,
             g0, g1, o0, o1):
        c = lax.axis_index("c")
        s = lax.axis_index("s")
        wid = s * NC + c
        row_base = wid * per_w
        tok_base = wid * tok_rows_per_w

        pltpu.sync_copy(tok_hbm.at[pl.ds(tok_base, tok_rows_per_w)], idx_all)

        def fire_g(ci, buf, sem):
            for j in range(NSUB):
                pltpu.async_copy(
                    table_hbm.at[idx_all.at[ci * NSUB + j]],
                    buf.at[pl.ds(j * SUB, SUB)],
                    sem,
                )

        def wait_g(buf, sem):
            pltpu.make_async_copy(table_hbm.at[pl.ds(0, CHUNK)], buf, sem).wait()

        def fire_o(ci, buf, sem):
            pltpu.async_copy(
                buf, out_hbm.at[pl.ds(row_base + ci * CHUNK, CHUNK)], sem
            )

        def wait_o(buf, sem):
            pltpu.make_async_copy(
                buf, out_hbm.at[pl.ds(row_base, CHUNK)], sem
            ).wait()

        def scale(buf):
            @plsc.parallel_loop(0, CHUNK, unroll=4)
            def _(r):
                for col in range(0, EMB, 16):
                    buf[r, pl.ds(col, 16)] = buf[r, pl.ds(col, 16)] * SCALE

        fire_g(0, rows0, g0)

        def pair(si, carry):
            c0 = 2 * si

            @pl.when(si > 0)
            def _():
                wait_o(rows1, o1)
            fire_g(c0 + 1, rows1, g1)
            wait_g(rows0, g0)
            scale(rows0)
            fire_o(c0, rows0, o0)

            wait_o(rows0, o0)
            @pl.when(si < n_pairs - 1)
            def _():
                fire_g(c0 + 2, rows0, g0)
            wait_g(rows1, g1)
            scale(rows1)
            fire_o(c0 + 1, rows1, o1)
            return carry

        lax.fori_loop(0, n_pairs, pair, 0)
        wait_o(rows1, o1)

    return body(tok2d, table)


def kernel(tokens, table):
    b, l = tokens.shape
    n = b * l
    tok2d = tokens.reshape(n // SUB, SUB).astype(jnp.int32)
    out = _lookup(tok2d, table, n)
    return out.reshape(b, l, EMB)
